# baseline (speedup 1.0000x reference)
import dataclasses
import functools

import jax
import jax.numpy as jnp
from jax import lax
from jax.experimental import pallas as pl
from jax.experimental.pallas import tpu as pltpu
from jax.experimental.pallas import tpu_sc as plsc

B = 16384
N_NUM = 13
F = 26
V = 100000
K = 16
NC = 2
NS = 16
BH = B // NC


def _sc_planes(vt3, lint3, idxt):
    mesh = plsc.VectorSubcoreMesh(core_axis_name="c", subcore_axis_name="s")
    cp = pltpu.CompilerParams()
    if "use_tc_tiling_on_sc" in pltpu.CompilerParams.__dataclass_fields__:
        cp = dataclasses.replace(cp, use_tc_tiling_on_sc=True)
    if "needs_layout_passes" in pltpu.CompilerParams.__dataclass_fields__:
        cp = dataclasses.replace(cp, needs_layout_passes=False)

    @functools.partial(
        pl.kernel,
        out_type=jax.ShapeDtypeStruct((NC * 3 * NS * BH,), jnp.float32),
        mesh=mesh,
        compiler_params=cp,
        scratch_types=[
            pltpu.VMEM((V,), jnp.float32),
            pltpu.VMEM((BH,), jnp.int32),
            pltpu.VMEM((BH,), jnp.float32),
            pltpu.VMEM((BH,), jnp.float32),
            pltpu.SemaphoreType.DMA,
            pltpu.SemaphoreType.DMA,
        ],
    )
    def k(vt_hbm, lf_hbm, idx_hbm, out_hbm, plane, idxf, acc, acc2,
          sem, sem2):
        c = lax.axis_index("c")
        t = lax.axis_index("s")
        bbase = c * BH

        @pl.loop(0, BH, step=16)
        def _(m):
            z = jnp.zeros((16,), jnp.float32)
            acc[pl.ds(m, 16)] = z
            acc2[pl.ds(m, 16)] = z

        @pl.loop(0, F)
        def _(f):
            cp = pltpu.async_copy(vt_hbm.at[f, t, :], plane, sem)
            ci = pltpu.async_copy(
                idx_hbm.at[pl.ds(f * B + bbase, BH)], idxf, sem2)
            cp.wait()
            ci.wait()

            @pl.loop(0, BH, step=64)
            def _(m0):
                for u in range(4):
                    m = m0 + u * 16
                    g = plsc.load_gather(plane, [idxf[pl.ds(m, 16)]])
                    acc[pl.ds(m, 16)] = acc[pl.ds(m, 16)] + g
                    acc2[pl.ds(m, 16)] = acc2[pl.ds(m, 16)] + g * g

        pltpu.sync_copy(acc, out_hbm.at[pl.ds(((c * 3 + 0) * NS + t) * BH, BH)])
        pltpu.sync_copy(acc2, out_hbm.at[pl.ds(((c * 3 + 1) * NS + t) * BH, BH)])

        @pl.loop(0, BH, step=16)
        def _(m):
            acc[pl.ds(m, 16)] = jnp.zeros((16,), jnp.float32)

        def lin_field(f):
            cp = pltpu.async_copy(lf_hbm.at[f, 0, :], plane, sem)
            ci = pltpu.async_copy(
                idx_hbm.at[pl.ds(f * B + bbase, BH)], idxf, sem2)
            cp.wait()
            ci.wait()

            @pl.loop(0, BH, step=64)
            def _(m0):
                for u in range(4):
                    m = m0 + u * 16
                    g = plsc.load_gather(plane, [idxf[pl.ds(m, 16)]])
                    acc[pl.ds(m, 16)] = acc[pl.ds(m, 16)] + g

        lin_field(t)

        @pl.when(t + NS < F)
        def _():
            lin_field(t + NS)

        pltpu.sync_copy(acc, out_hbm.at[pl.ds(((c * 3 + 2) * NS + t) * BH, BH)])

    return k(vt3, lint3, idxt)


def _combine(scout, x_num, v_num, w_row, const):
    BLK = 512
    NBH = BH // BLK

    def body(sc_ref, x_ref, vn_ref, w_ref, c_ref, o_ref):
        sc = sc_ref[0]
        sv_cat = jnp.transpose(sc[0])
        sq_cat = jnp.transpose(sc[1])
        lp = jnp.transpose(sc[2])
        x = x_ref[...]
        vn = vn_ref[...]
        sv = sv_cat + jnp.dot(x, vn, preferred_element_type=jnp.float32)
        sq = sq_cat + jnp.dot(x * x, vn * vn,
                              preferred_element_type=jnp.float32)
        lin = (jnp.sum(lp, axis=1, keepdims=True)
               + jnp.sum(x * w_ref[...], axis=1, keepdims=True)
               + c_ref[0, 0])
        o_ref[...] = lin + 0.5 * jnp.sum(sv * sv - sq, axis=1, keepdims=True)

    return pl.pallas_call(
        body,
        grid=(B // BLK,),
        in_specs=[
            pl.BlockSpec((1, 3, NS, BLK), lambda i: (i // NBH, 0, 0, i % NBH)),
            pl.BlockSpec((BLK, N_NUM), lambda i: (i, 0)),
            pl.BlockSpec((N_NUM, K), lambda i: (0, 0)),
            pl.BlockSpec((1, N_NUM), lambda i: (0, 0)),
            pl.BlockSpec((1, 1), lambda i: (0, 0)),
        ],
        out_specs=pl.BlockSpec((BLK, 1), lambda i: (i, 0)),
        out_shape=jax.ShapeDtypeStruct((B, 1), jnp.float32),
    )(scout, x_num, v_num, w_row, const)


def kernel(x_num, x_cat, bias, W_num, lin_cat, v_num, v_cat):
    xc = x_cat.astype(jnp.int32)
    idxt = jnp.transpose(xc).reshape(-1)
    vt3 = jnp.transpose(v_cat, (0, 2, 1))
    lint3 = jnp.transpose(lin_cat, (0, 2, 1))
    scout = _sc_planes(vt3, lint3, idxt).reshape(NC, 3, NS, BH)
    const = bias.reshape(1, 1)
    return _combine(scout, x_num, v_num, W_num, const)
